# baseline (speedup 1.0000x reference)
import jax
import jax.numpy as jnp
from jax import lax
from jax.experimental import pallas as pl
from jax.experimental.pallas import tpu as pltpu

_B, _C, _H, _W = 64, 3, 224, 224
_P, _GH, _GW, _D = 14, 16, 16, 768
_MAXT = 10
_NP = _GH * _GW
_K = _C * _P * _P
_PATCH_H = _H / _GH
_PATCH_W = _W / _GW


def _fused_body(img_ref, boxes_ref, wp_ref, wa_ref, cls_ref, obj_ref,
                attn_ref):
    wp = wp_ref[...]
    wa = wa_ref[...]
    wc = jnp.dot(wp, wa, preferred_element_type=jnp.float32)
    wc = wc.astype(jnp.bfloat16)

    slabs = []
    for c in range(_C):
        for i in range(_P):
            sl = img_ref[0, c, :, i, :, :].reshape(_NP, _P)
            slabs.append(sl.astype(jnp.bfloat16))

    logits_col = jnp.zeros((_NP, 1), jnp.float32)
    for ci in range(_C * _P):
        logits_col += jnp.dot(slabs[ci], wc[ci * _P:(ci + 1) * _P],
                              preferred_element_type=jnp.float32)

    bx = boxes_ref[0]
    x0 = jnp.clip(bx[:, 0] * _W, 0.0, float(_W))
    y0 = jnp.clip(bx[:, 1] * _H, 0.0, float(_H))
    x1 = jnp.clip(bx[:, 2] * _W, 0.0, float(_W))
    y1 = jnp.clip(bx[:, 3] * _H, 0.0, float(_H))
    x0i = jnp.clip(jnp.floor(x0 / _PATCH_W).astype(jnp.int32), 0, _GW - 1)
    y0i = jnp.clip(jnp.floor(y0 / _PATCH_H).astype(jnp.int32), 0, _GH - 1)
    x1i = jnp.clip(jnp.ceil(x1 / _PATCH_W).astype(jnp.int32), x0i + 1, _GW)
    y1i = jnp.clip(jnp.ceil(y1 / _PATCH_H).astype(jnp.int32), y0i + 1, _GH)

    p_ids = lax.broadcasted_iota(jnp.int32, (_NP, _MAXT), 0)
    gy = p_ids // _GW
    gx = p_ids % _GW
    maskT = ((gy >= y0i[None, :]) & (gy < y1i[None, :]) &
             (gx >= x0i[None, :]) & (gx < x1i[None, :]))

    neg = jnp.float32(-1e30)
    mlT = jnp.where(maskT, logits_col, neg)
    mlT = mlT - jnp.max(mlT, axis=0, keepdims=True)
    ewT = jnp.exp(mlT)
    ewT = jnp.where(maskT, ewT, 0.0)
    wT = ewT / jnp.sum(ewT, axis=0, keepdims=True)

    vcatT = jnp.concatenate(
        [wT, jnp.full((_NP, 1), 1.0 / _NP, jnp.float32)], 1)
    vcatT16 = vcatT.astype(jnp.bfloat16)

    zparts = [lax.dot_general(vcatT16, slabs[ci], (((0,), (0,)), ((), ())),
                              preferred_element_type=jnp.float32)
              for ci in range(_C * _P)]
    zcat = jnp.concatenate(zparts, axis=1)
    zcat16 = zcat.astype(jnp.bfloat16)
    out11 = jnp.dot(zcat16, wp, preferred_element_type=jnp.float32)

    obj_ref[0] = out11[:_MAXT]
    cls_ref[0] = out11[_MAXT:]
    attn_ref[0] = wT.T


def kernel(images, boxes, scores, W_patch, W_att, b_att):
    img6 = images.reshape(_B, _C, _GH, _P, _GW, _P)
    wp16 = W_patch.astype(jnp.bfloat16)
    wa16 = W_att.astype(jnp.bfloat16)

    cls_tokens, object_tokens, attention_maps = pl.pallas_call(
        _fused_body,
        grid=(_B,),
        in_specs=[
            pl.BlockSpec((1, _C, _GH, _P, _GW, _P),
                         lambda b: (b, 0, 0, 0, 0, 0)),
            pl.BlockSpec((1, _MAXT, 4), lambda b: (b, 0, 0)),
            pl.BlockSpec((_K, _D), lambda b: (0, 0)),
            pl.BlockSpec((_D, 1), lambda b: (0, 0)),
        ],
        out_specs=[
            pl.BlockSpec((1, 1, _D), lambda b: (b, 0, 0)),
            pl.BlockSpec((1, _MAXT, _D), lambda b: (b, 0, 0)),
            pl.BlockSpec((1, _MAXT, _NP), lambda b: (b, 0, 0)),
        ],
        out_shape=[
            jax.ShapeDtypeStruct((_B, 1, _D), jnp.float32),
            jax.ShapeDtypeStruct((_B, _MAXT, _D), jnp.float32),
            jax.ShapeDtypeStruct((_B, _MAXT, _NP), jnp.float32),
        ],
    )(img6, boxes, wp16, wa16)

    object_mask = jnp.ones((_B, _MAXT), dtype=bool)
    return (cls_tokens.reshape(_B, _D), object_tokens, object_mask, boxes,
            scores, attention_maps)


# speedup vs baseline: 1.0111x; 1.0111x over previous
import jax
import jax.numpy as jnp
from jax import lax
from jax.experimental import pallas as pl
from jax.experimental.pallas import tpu as pltpu

_B, _C, _H, _W = 64, 3, 224, 224
_P, _GH, _GW, _D = 14, 16, 16, 768
_MAXT = 10
_NP = _GH * _GW
_K = _C * _P * _P
_PATCH_H = _H / _GH
_PATCH_W = _W / _GW


def _fused_body(img_ref, boxes_ref, wp_ref, wa_ref, cls_ref, obj_ref,
                attn_ref):
    wp = wp_ref[...]
    wa = wa_ref[...]
    wc = jnp.dot(wp, wa, preferred_element_type=jnp.float32)
    wc = wc.astype(jnp.bfloat16)

    slabs = []
    for c in range(_C):
        for i in range(_P):
            sl = img_ref[0, c, :, i, :, :].reshape(_NP, _P)
            slabs.append(sl.astype(jnp.bfloat16))

    logits = jnp.zeros((1, _NP), jnp.float32)
    for ci in range(_C * _P):
        logits += lax.dot_general(
            wc[ci * _P:(ci + 1) * _P], slabs[ci], (((0,), (1,)), ((), ())),
            preferred_element_type=jnp.float32)

    bx = boxes_ref[0]
    x0 = jnp.clip(bx[:, 0] * _W, 0.0, float(_W))
    y0 = jnp.clip(bx[:, 1] * _H, 0.0, float(_H))
    x1 = jnp.clip(bx[:, 2] * _W, 0.0, float(_W))
    y1 = jnp.clip(bx[:, 3] * _H, 0.0, float(_H))
    x0i = jnp.clip(jnp.floor(x0 / _PATCH_W).astype(jnp.int32), 0, _GW - 1)
    y0i = jnp.clip(jnp.floor(y0 / _PATCH_H).astype(jnp.int32), 0, _GH - 1)
    x1i = jnp.clip(jnp.ceil(x1 / _PATCH_W).astype(jnp.int32), x0i + 1, _GW)
    y1i = jnp.clip(jnp.ceil(y1 / _PATCH_H).astype(jnp.int32), y0i + 1, _GH)

    p_ids = lax.broadcasted_iota(jnp.int32, (_MAXT, _NP), 1)
    gy = p_ids // _GW
    gx = p_ids % _GW
    mask = ((gy >= y0i[:, None]) & (gy < y1i[:, None]) &
            (gx >= x0i[:, None]) & (gx < x1i[:, None]))

    neg = jnp.float32(-1e30)
    ml = jnp.where(mask, logits, neg)
    ml = ml - jnp.max(ml, axis=-1, keepdims=True)
    ew = jnp.exp(ml)
    ew = jnp.where(mask, ew, 0.0)
    w = ew / jnp.sum(ew, axis=-1, keepdims=True)

    vcat = jnp.concatenate(
        [w, jnp.full((1, _NP), 1.0 / _NP, jnp.float32)], 0)
    vcat16 = vcat.astype(jnp.bfloat16)

    zparts = [jnp.dot(vcat16, slabs[ci],
                      preferred_element_type=jnp.float32)
              for ci in range(_C * _P)]
    zcat = jnp.concatenate(zparts, axis=1)
    zcat16 = zcat.astype(jnp.bfloat16)
    out11 = jnp.dot(zcat16, wp, preferred_element_type=jnp.float32)

    obj_ref[0] = out11[:_MAXT]
    cls_ref[0] = out11[_MAXT:]
    attn_ref[0] = w


def kernel(images, boxes, scores, W_patch, W_att, b_att):
    img6 = images.reshape(_B, _C, _GH, _P, _GW, _P)
    wp16 = W_patch.astype(jnp.bfloat16)
    wa16 = W_att.astype(jnp.bfloat16)

    cls_tokens, object_tokens, attention_maps = pl.pallas_call(
        _fused_body,
        grid=(_B,),
        in_specs=[
            pl.BlockSpec((1, _C, _GH, _P, _GW, _P),
                         lambda b: (b, 0, 0, 0, 0, 0)),
            pl.BlockSpec((1, _MAXT, 4), lambda b: (b, 0, 0)),
            pl.BlockSpec((_K, _D), lambda b: (0, 0)),
            pl.BlockSpec((_D, 1), lambda b: (0, 0)),
        ],
        out_specs=[
            pl.BlockSpec((1, 1, _D), lambda b: (b, 0, 0)),
            pl.BlockSpec((1, _MAXT, _D), lambda b: (b, 0, 0)),
            pl.BlockSpec((1, _MAXT, _NP), lambda b: (b, 0, 0)),
        ],
        out_shape=[
            jax.ShapeDtypeStruct((_B, 1, _D), jnp.float32),
            jax.ShapeDtypeStruct((_B, _MAXT, _D), jnp.float32),
            jax.ShapeDtypeStruct((_B, _MAXT, _NP), jnp.float32),
        ],
    )(img6, boxes, wp16, wa16)

    object_mask = jnp.ones((_B, _MAXT), dtype=bool)
    return (cls_tokens.reshape(_B, _D), object_tokens, object_mask, boxes,
            scores, attention_maps)


# speedup vs baseline: 1.6712x; 1.6528x over previous
import jax
import jax.numpy as jnp
from jax import lax
from jax.experimental import pallas as pl
from jax.experimental.pallas import tpu as pltpu

_B, _C, _H, _W = 64, 3, 224, 224
_P, _GH, _GW, _D = 14, 16, 16, 768
_MAXT = 10
_NP = _GH * _GW
_K = _C * _P * _P
_PATCH_H = _H / _GH
_PATCH_W = _W / _GW
_BB = 8


def _fused_body(xt_ref, boxes_ref, wp_ref, wa_ref, cls_ref, obj_ref, attn_ref):
    wp = wp_ref[...]
    wa = wa_ref[...]
    wc = jnp.dot(wp, wa, preferred_element_type=jnp.float32)
    wc = wc.astype(jnp.bfloat16)

    for i in range(_BB):
        xt = xt_ref[i]
        bx = boxes_ref[i]

        logits = lax.dot_general(wc, xt, (((0,), (1,)), ((), ())),
                                 preferred_element_type=jnp.float32)

        x0 = jnp.clip(bx[:, 0] * _W, 0.0, float(_W))
        y0 = jnp.clip(bx[:, 1] * _H, 0.0, float(_H))
        x1 = jnp.clip(bx[:, 2] * _W, 0.0, float(_W))
        y1 = jnp.clip(bx[:, 3] * _H, 0.0, float(_H))
        x0i = jnp.clip(jnp.floor(x0 / _PATCH_W).astype(jnp.int32), 0, _GW - 1)
        y0i = jnp.clip(jnp.floor(y0 / _PATCH_H).astype(jnp.int32), 0, _GH - 1)
        x1i = jnp.clip(jnp.ceil(x1 / _PATCH_W).astype(jnp.int32), x0i + 1, _GW)
        y1i = jnp.clip(jnp.ceil(y1 / _PATCH_H).astype(jnp.int32), y0i + 1, _GH)

        p_ids = lax.broadcasted_iota(jnp.int32, (_MAXT, _NP), 1)
        gy = p_ids // _GW
        gx = p_ids % _GW
        mask = ((gy >= y0i[:, None]) & (gy < y1i[:, None]) &
                (gx >= x0i[:, None]) & (gx < x1i[:, None]))

        neg = jnp.float32(-1e30)
        ml = jnp.where(mask, logits, neg)
        ml = ml - jnp.max(ml, axis=-1, keepdims=True)
        ew = jnp.exp(ml)
        ew = jnp.where(mask, ew, 0.0)
        w = ew / jnp.sum(ew, axis=-1, keepdims=True)

        vcat = jnp.concatenate(
            [w, jnp.full((1, _NP), 1.0 / _NP, jnp.float32)], 0)
        vcat16 = vcat.astype(jnp.bfloat16)
        zcat = jnp.dot(vcat16, xt, preferred_element_type=jnp.float32)
        zcat16 = zcat.astype(jnp.bfloat16)
        out11 = jnp.dot(zcat16, wp, preferred_element_type=jnp.float32)

        obj_ref[i] = out11[:_MAXT]
        cls_ref[i] = out11[_MAXT:]
        attn_ref[i] = w


def kernel(images, boxes, scores, W_patch, W_att, b_att):
    xt = images.astype(jnp.bfloat16).reshape(_B, _C, _GH, _P, _GW, _P)
    xt = xt.transpose(0, 2, 4, 1, 3, 5).reshape(_B, _NP, _K)
    wp16 = W_patch.astype(jnp.bfloat16)
    wa16 = W_att.astype(jnp.bfloat16)

    nb = _B // _BB
    cls_tokens, object_tokens, attention_maps = pl.pallas_call(
        _fused_body,
        grid=(nb,),
        in_specs=[
            pl.BlockSpec((_BB, _NP, _K), lambda b: (b, 0, 0)),
            pl.BlockSpec((_BB, _MAXT, 4), lambda b: (b, 0, 0)),
            pl.BlockSpec((_K, _D), lambda b: (0, 0)),
            pl.BlockSpec((_D, 1), lambda b: (0, 0)),
        ],
        out_specs=[
            pl.BlockSpec((_BB, 1, _D), lambda b: (b, 0, 0)),
            pl.BlockSpec((_BB, _MAXT, _D), lambda b: (b, 0, 0)),
            pl.BlockSpec((_BB, _MAXT, _NP), lambda b: (b, 0, 0)),
        ],
        out_shape=[
            jax.ShapeDtypeStruct((_B, 1, _D), jnp.float32),
            jax.ShapeDtypeStruct((_B, _MAXT, _D), jnp.float32),
            jax.ShapeDtypeStruct((_B, _MAXT, _NP), jnp.float32),
        ],
    )(xt, boxes, wp16, wa16)

    object_mask = jnp.ones((_B, _MAXT), dtype=bool)
    return (cls_tokens.reshape(_B, _D), object_tokens, object_mask, boxes,
            scores, attention_maps)


# speedup vs baseline: 1.6718x; 1.0004x over previous
import jax
import jax.numpy as jnp
from jax import lax
from jax.experimental import pallas as pl
from jax.experimental.pallas import tpu as pltpu

_B, _C, _H, _W = 64, 3, 224, 224
_P, _GH, _GW, _D = 14, 16, 16, 768
_MAXT = 10
_NP = _GH * _GW
_K = _C * _P * _P
_PATCH_H = _H / _GH
_PATCH_W = _W / _GW
_BB = 8


def _fused_body(xt_ref, boxes_ref, wp_ref, wa_ref, cls_ref, obj_ref, attn_ref):
    wp = wp_ref[...]
    wa = wa_ref[...]
    wc = jnp.dot(wp, wa, preferred_element_type=jnp.float32)
    wc = wc.astype(jnp.bfloat16)

    for i in range(_BB):
        xt = xt_ref[i]
        bx = boxes_ref[i]

        logits = lax.dot_general(wc, xt, (((0,), (1,)), ((), ())),
                                 preferred_element_type=jnp.float32)

        x0 = jnp.clip(bx[:, 0] * _W, 0.0, float(_W))
        y0 = jnp.clip(bx[:, 1] * _H, 0.0, float(_H))
        x1 = jnp.clip(bx[:, 2] * _W, 0.0, float(_W))
        y1 = jnp.clip(bx[:, 3] * _H, 0.0, float(_H))
        x0i = jnp.clip(jnp.floor(x0 / _PATCH_W).astype(jnp.int32), 0, _GW - 1)
        y0i = jnp.clip(jnp.floor(y0 / _PATCH_H).astype(jnp.int32), 0, _GH - 1)
        x1i = jnp.clip(jnp.ceil(x1 / _PATCH_W).astype(jnp.int32), x0i + 1, _GW)
        y1i = jnp.clip(jnp.ceil(y1 / _PATCH_H).astype(jnp.int32), y0i + 1, _GH)

        p_ids = lax.broadcasted_iota(jnp.int32, (_MAXT, _NP), 1)
        gy = p_ids // _GW
        gx = p_ids % _GW
        mask = ((gy >= y0i[:, None]) & (gy < y1i[:, None]) &
                (gx >= x0i[:, None]) & (gx < x1i[:, None]))

        neg = jnp.float32(-1e30)
        ml = jnp.where(mask, logits, neg)
        ml = ml - jnp.max(ml, axis=-1, keepdims=True)
        ew = jnp.exp(ml)
        ew = jnp.where(mask, ew, 0.0)
        w = ew / jnp.sum(ew, axis=-1, keepdims=True)

        vcat = jnp.concatenate(
            [w, jnp.full((1, _NP), 1.0 / _NP, jnp.float32)], 0)
        vcat16 = vcat.astype(jnp.bfloat16)
        zcat = jnp.dot(vcat16, xt, preferred_element_type=jnp.float32)
        zcat16 = zcat.astype(jnp.bfloat16)
        out11 = jnp.dot(zcat16, wp, preferred_element_type=jnp.float32)

        obj_ref[i] = out11[:_MAXT]
        cls_ref[i] = out11[_MAXT:]
        attn_ref[i] = w


def kernel(images, boxes, scores, W_patch, W_att, b_att):
    img6 = images.astype(jnp.bfloat16).reshape(_B, _C, _GH, _P, _GW, _P)
    xt = lax.reshape(img6, (_B, _NP, _K), dimensions=(0, 2, 4, 1, 3, 5))
    wp16 = W_patch.astype(jnp.bfloat16)
    wa16 = W_att.astype(jnp.bfloat16)

    nb = _B // _BB
    cls_tokens, object_tokens, attention_maps = pl.pallas_call(
        _fused_body,
        grid=(nb,),
        in_specs=[
            pl.BlockSpec((_BB, _NP, _K), lambda b: (b, 0, 0)),
            pl.BlockSpec((_BB, _MAXT, 4), lambda b: (b, 0, 0)),
            pl.BlockSpec((_K, _D), lambda b: (0, 0)),
            pl.BlockSpec((_D, 1), lambda b: (0, 0)),
        ],
        out_specs=[
            pl.BlockSpec((_BB, 1, _D), lambda b: (b, 0, 0)),
            pl.BlockSpec((_BB, _MAXT, _D), lambda b: (b, 0, 0)),
            pl.BlockSpec((_BB, _MAXT, _NP), lambda b: (b, 0, 0)),
        ],
        out_shape=[
            jax.ShapeDtypeStruct((_B, 1, _D), jnp.float32),
            jax.ShapeDtypeStruct((_B, _MAXT, _D), jnp.float32),
            jax.ShapeDtypeStruct((_B, _MAXT, _NP), jnp.float32),
        ],
    )(xt, boxes, wp16, wa16)

    object_mask = jnp.ones((_B, _MAXT), dtype=bool)
    return (cls_tokens.reshape(_B, _D), object_tokens, object_mask, boxes,
            scores, attention_maps)
